# baseline (speedup 1.0000x reference)
import functools

import jax
import jax.numpy as jnp
from jax import lax
from jax.experimental import pallas as pl
from jax.experimental.pallas import tpu as pltpu
from jax.experimental.pallas import tpu_sc as plsc

N_NODE = 10000
N_PAD = 10240
DIM = 256
HALF = 128
K = 112
N_CHUNK = 90
E_TILE = N_CHUNK * K
N_TILE = 16
N_CORE = 2
ROWS_TILE = N_PAD // N_TILE
FB = 80


def _sc_mesh():
    return plsc.VectorSubcoreMesh(core_axis_name="c", subcore_axis_name="s")


def _lane_splat(vec, lane):
    idx = jnp.full((16, 1), lane, jnp.int32)
    return lax.gather(
        vec, idx,
        dimension_numbers=lax.GatherDimensionNumbers(
            offset_dims=(), collapsed_slice_dims=(0,), start_index_map=(0,)),
        slice_sizes=(1,),
        mode=lax.GatherScatterMode.PROMISE_IN_BOUNDS)


@functools.partial(
    pl.kernel,
    out_type=jax.ShapeDtypeStruct((N_CORE, N_PAD), jnp.float32),
    mesh=_sc_mesh(),
    scratch_types=[
        pltpu.VMEM((N_CHUNK, K), jnp.int32),
        pltpu.VMEM((N_CHUNK, K), jnp.float32),
        pltpu.VMEM((ROWS_TILE,), jnp.float32),
        pltpu.VMEM_SHARED((N_PAD,), jnp.float32),
    ],
)
def _deg_kernel(src_hbm, ew_hbm, out_hbm, src_v, ew_v, zero_v, acc):
    c = lax.axis_index("c")
    s = lax.axis_index("s")

    def zbody(i, carry):
        zero_v[pl.ds(i * 16, 16)] = jnp.zeros((16,), jnp.float32)
        return carry

    lax.fori_loop(0, ROWS_TILE // 16, zbody, 0)
    pltpu.sync_copy(zero_v, acc.at[pl.ds(s * ROWS_TILE, ROWS_TILE)])
    plsc.subcore_barrier()

    pltpu.sync_copy(src_hbm.at[s], src_v)
    pltpu.sync_copy(ew_hbm.at[s], ew_v)

    def body(i, carry):
        j = c * (N_CHUNK // 2) + i
        pltpu.sync_copy(ew_v.at[j], acc.at[src_v.at[j]], add=True)
        return carry

    lax.fori_loop(0, N_CHUNK // 2, body, 0)
    plsc.subcore_barrier()
    pltpu.sync_copy(acc.at[pl.ds(s * ROWS_TILE, ROWS_TILE)],
                    out_hbm.at[c, pl.ds(s * ROWS_TILE, ROWS_TILE)])


def _mm_body(x_ref, w_ref, deg_ref, h2_ref):
    d = deg_ref[0, :] + deg_ref[1, :] + 1.0
    dinv = lax.rsqrt(d)
    h = jnp.dot(x_ref[...], w_ref[...], preferred_element_type=jnp.float32)
    h2_ref[...] = (h * dinv[:, None])[None]


def _matmul_scaled(x_pad, W, degp):
    blk = 1024
    return pl.pallas_call(
        _mm_body,
        grid=(N_PAD // blk, 2),
        in_specs=[
            pl.BlockSpec((blk, DIM), lambda i, k: (i, 0)),
            pl.BlockSpec((DIM, HALF), lambda i, k: (0, k)),
            pl.BlockSpec((N_CORE, blk), lambda i, k: (0, i)),
        ],
        out_specs=pl.BlockSpec((1, blk, HALF), lambda i, k: (k, i, 0)),
        out_shape=jax.ShapeDtypeStruct((N_CORE, N_PAD, HALF), jnp.float32),
    )(x_pad, W, degp)


@functools.partial(
    pl.kernel,
    out_type=jax.ShapeDtypeStruct((N_PAD, DIM), jnp.float32),
    mesh=_sc_mesh(),
    scratch_types=[
        pltpu.VMEM((3, K), jnp.int32),
        pltpu.VMEM((3, K), jnp.float32),
        pltpu.VMEM((3, K), jnp.int32),
        pltpu.VMEM((3, K), jnp.int32),
        pltpu.VMEM((3, K, HALF), jnp.float32),
        pltpu.VMEM((HALF,), jnp.float32),
        pltpu.VMEM_SHARED((N_PAD, HALF), jnp.float32),
        [pltpu.SemaphoreType.DMA] * 3,
        [pltpu.SemaphoreType.DMA] * 3,
        [pltpu.SemaphoreType.DMA] * 3,
    ],
)
def _agg_kernel(h2_hbm, src_hbm, ew_hbm, dst_hbm, b_hbm, out_hbm,
                idxc, ewc, dstc, sdst, rows_v, b_v, acc, gsem, psem, ssem):
    c = lax.axis_index("c")
    s = lax.axis_index("s")

    def zb(r, carry):
        for g in range(HALF // 16):
            rows_v[0, r, pl.ds(g * 16, 16)] = jnp.zeros((16,), jnp.float32)
        return carry

    lax.fori_loop(0, FB, zb, 0)
    for kk in range(ROWS_TILE // FB):
        pltpu.sync_copy(rows_v.at[0, pl.ds(0, FB)],
                        acc.at[pl.ds(s * ROWS_TILE + kk * FB, FB)])
    plsc.subcore_barrier()

    def prefetch(j, m):
        pltpu.async_copy(src_hbm.at[s, j], idxc.at[m], psem[m])
        pltpu.async_copy(ew_hbm.at[s, j], ewc.at[m], psem[m])
        pltpu.async_copy(dst_hbm.at[s, j], dstc.at[m], psem[m])

    def wait_prefetch(j, m):
        pltpu.make_async_copy(src_hbm.at[s, j], idxc.at[m], psem[m]).wait()
        pltpu.make_async_copy(ew_hbm.at[s, j], ewc.at[m], psem[m]).wait()
        pltpu.make_async_copy(dst_hbm.at[s, j], dstc.at[m], psem[m]).wait()

    def issue_gather(m):
        pltpu.async_copy(h2_hbm.at[c].at[idxc.at[m]], rows_v.at[m], gsem[m])

    def wait_gather(m):
        pltpu.make_async_copy(h2_hbm.at[c].at[idxc.at[m]], rows_v.at[m],
                              gsem[m]).wait()

    def wait_scatter(m):
        pltpu.make_async_copy(rows_v.at[m], acc.at[sdst.at[m]],
                              ssem[m]).wait()

    prefetch(0, 0)
    prefetch(1, 1)
    wait_prefetch(0, 0)
    issue_gather(0)

    def ring(i, carry):
        for b in range(3):
            j = 3 * i + b
            m = b
            m1 = (b + 1) % 3
            m2 = (b + 2) % 3
            j1 = j + 1
            j2 = j + 2

            @pl.when(j2 < N_CHUNK)
            def _():
                prefetch(j2, m2)

            @pl.when(j1 < N_CHUNK)
            def _():
                @pl.when(j1 >= 3)
                def _():
                    wait_scatter(m1)

                wait_prefetch(j1, m1)
                issue_gather(m1)

            wait_gather(m)
            for g in range(K // 16):
                sdst[m, pl.ds(g * 16, 16)] = dstc[m, pl.ds(g * 16, 16)]

            def sg(g, carry2):
                wg = ewc[m, pl.ds(g * 16, 16)]
                for l in range(16):
                    e = g * 16 + l
                    wv = _lane_splat(wg, l)
                    for gg in range(HALF // 16):
                        rows_v[m, e, pl.ds(gg * 16, 16)] = (
                            rows_v[m, e, pl.ds(gg * 16, 16)] * wv)
                return carry2

            lax.fori_loop(0, K // 16, sg, 0)
            pltpu.async_copy(rows_v.at[m], acc.at[sdst.at[m]], ssem[m],
                             add=True)
        return carry

    lax.fori_loop(0, N_CHUNK // 3, ring, 0)
    wait_scatter((N_CHUNK - 3) % 3)
    wait_scatter((N_CHUNK - 2) % 3)
    wait_scatter((N_CHUNK - 1) % 3)
    plsc.subcore_barrier()

    pltpu.sync_copy(b_hbm.at[pl.ds(c * HALF, HALF)], b_v)
    for kk in range(ROWS_TILE // FB):
        row0 = s * ROWS_TILE + kk * FB
        pltpu.sync_copy(acc.at[pl.ds(row0, FB)], rows_v.at[0, pl.ds(0, FB)])

        def fb(r, carry):
            for g in range(HALF // 16):
                v = (rows_v[0, r, pl.ds(g * 16, 16)]
                     + b_v[pl.ds(g * 16, 16)])
                rows_v[0, r, pl.ds(g * 16, 16)] = jnp.maximum(v, 0.0)
            return carry

        lax.fori_loop(0, FB, fb, 0)
        pltpu.sync_copy(rows_v.at[0, pl.ds(0, FB)],
                        out_hbm.at[pl.ds(row0, FB), pl.ds(c * HALF, HALF)])


def kernel(x, edge_index, edge_weight, W, b):
    src = edge_index[0]
    dst = edge_index[1]
    n_edges = src.shape[0]
    e_pad = N_TILE * E_TILE
    pad = e_pad - n_edges
    src_g = jnp.pad(src, (0, pad)).reshape(N_TILE, N_CHUNK, K)
    dst_g = jnp.pad(dst, (0, pad)).reshape(N_TILE, N_CHUNK, K)
    ew_g = jnp.pad(edge_weight, (0, pad)).reshape(N_TILE, N_CHUNK, K)
    x_pad = jnp.pad(x, ((0, N_PAD - x.shape[0]), (0, 0)))

    degp = _deg_kernel(src_g, ew_g)
    h2 = _matmul_scaled(x_pad, W, degp)
    out_pad = _agg_kernel(h2, src_g, ew_g, dst_g, b)
    return out_pad[:N_NODE]


# speedup vs baseline: 1.0392x; 1.0392x over previous
import functools

import jax
import jax.numpy as jnp
from jax import lax
from jax.experimental import pallas as pl
from jax.experimental.pallas import tpu as pltpu
from jax.experimental.pallas import tpu_sc as plsc

N_NODE = 10000
N_PAD = 10240
DIM = 256
HALF = 128
K = 112
N_CHUNK = 90
E_TILE = N_CHUNK * K
N_TILE = 16
N_CORE = 2
ROWS_TILE = N_PAD // N_TILE
FB = 80


def _sc_mesh():
    return plsc.VectorSubcoreMesh(core_axis_name="c", subcore_axis_name="s")


def _lane_splat(vec, lane):
    idx = jnp.full((16, 1), lane, jnp.int32)
    return lax.gather(
        vec, idx,
        dimension_numbers=lax.GatherDimensionNumbers(
            offset_dims=(), collapsed_slice_dims=(0,), start_index_map=(0,)),
        slice_sizes=(1,),
        mode=lax.GatherScatterMode.PROMISE_IN_BOUNDS)


@functools.partial(
    pl.kernel,
    out_type=jax.ShapeDtypeStruct((N_CORE, N_PAD), jnp.float32),
    mesh=_sc_mesh(),
    scratch_types=[
        pltpu.VMEM((N_CHUNK, K), jnp.int32),
        pltpu.VMEM((N_CHUNK, K), jnp.float32),
        pltpu.VMEM((ROWS_TILE,), jnp.float32),
        pltpu.VMEM_SHARED((N_PAD,), jnp.float32),
    ],
)
def _deg_kernel(src_hbm, ew_hbm, out_hbm, src_v, ew_v, zero_v, acc):
    c = lax.axis_index("c")
    s = lax.axis_index("s")

    def zbody(i, carry):
        zero_v[pl.ds(i * 16, 16)] = jnp.zeros((16,), jnp.float32)
        return carry

    lax.fori_loop(0, ROWS_TILE // 16, zbody, 0)
    pltpu.sync_copy(zero_v, acc.at[pl.ds(s * ROWS_TILE, ROWS_TILE)])
    plsc.subcore_barrier()

    pltpu.sync_copy(src_hbm.at[s], src_v)
    pltpu.sync_copy(ew_hbm.at[s], ew_v)

    def body(i, carry):
        j = c * (N_CHUNK // 2) + i
        pltpu.sync_copy(ew_v.at[j], acc.at[src_v.at[j]], add=True)
        return carry

    lax.fori_loop(0, N_CHUNK // 2, body, 0)
    plsc.subcore_barrier()
    pltpu.sync_copy(acc.at[pl.ds(s * ROWS_TILE, ROWS_TILE)],
                    out_hbm.at[c, pl.ds(s * ROWS_TILE, ROWS_TILE)])


def _mm_body(x_ref, w_ref, deg_ref, h2_ref):
    d = deg_ref[0, :] + deg_ref[1, :] + 1.0
    dinv = lax.rsqrt(d)
    h = jnp.dot(x_ref[...], w_ref[...], preferred_element_type=jnp.float32)
    h2_ref[...] = (h * dinv[:, None])[None]


def _matmul_scaled(x_pad, W, degp):
    blk = 1024
    return pl.pallas_call(
        _mm_body,
        grid=(N_PAD // blk, 2),
        in_specs=[
            pl.BlockSpec((blk, DIM), lambda i, k: (i, 0)),
            pl.BlockSpec((DIM, HALF), lambda i, k: (0, k)),
            pl.BlockSpec((N_CORE, blk), lambda i, k: (0, i)),
        ],
        out_specs=pl.BlockSpec((1, blk, HALF), lambda i, k: (k, i, 0)),
        out_shape=jax.ShapeDtypeStruct((N_CORE, N_PAD, HALF), jnp.float32),
    )(x_pad, W, degp)


@functools.partial(
    pl.kernel,
    out_type=jax.ShapeDtypeStruct((N_NODE, DIM), jnp.float32),
    mesh=_sc_mesh(),
    scratch_types=[
        pltpu.VMEM((3, K), jnp.int32),
        pltpu.VMEM((3, K), jnp.float32),
        pltpu.VMEM((3, K), jnp.int32),
        pltpu.VMEM((3, K), jnp.int32),
        pltpu.VMEM((3, K, HALF), jnp.float32),
        pltpu.VMEM((HALF,), jnp.float32),
        pltpu.VMEM_SHARED((N_PAD, HALF), jnp.float32),
        [pltpu.SemaphoreType.DMA] * 3,
        [pltpu.SemaphoreType.DMA] * 3,
        [pltpu.SemaphoreType.DMA] * 3,
    ],
)
def _agg_kernel(h2_hbm, src_hbm, ew_hbm, dst_hbm, b_hbm, out_hbm,
                idxc, ewc, dstc, sdst, rows_v, b_v, acc, gsem, psem, ssem):
    c = lax.axis_index("c")
    s = lax.axis_index("s")

    def zb(r, carry):
        for g in range(HALF // 16):
            rows_v[0, r, pl.ds(g * 16, 16)] = jnp.zeros((16,), jnp.float32)
        return carry

    lax.fori_loop(0, FB, zb, 0)
    for kk in range(ROWS_TILE // FB):
        pltpu.sync_copy(rows_v.at[0, pl.ds(0, FB)],
                        acc.at[pl.ds(s * ROWS_TILE + kk * FB, FB)])
    plsc.subcore_barrier()

    def prefetch(j, m):
        pltpu.async_copy(src_hbm.at[s, j], idxc.at[m], psem[m])
        pltpu.async_copy(ew_hbm.at[s, j], ewc.at[m], psem[m])
        pltpu.async_copy(dst_hbm.at[s, j], dstc.at[m], psem[m])

    def wait_prefetch(j, m):
        pltpu.make_async_copy(src_hbm.at[s, j], idxc.at[m], psem[m]).wait()
        pltpu.make_async_copy(ew_hbm.at[s, j], ewc.at[m], psem[m]).wait()
        pltpu.make_async_copy(dst_hbm.at[s, j], dstc.at[m], psem[m]).wait()

    def issue_gather(m):
        pltpu.async_copy(h2_hbm.at[c].at[idxc.at[m]], rows_v.at[m], gsem[m])

    def wait_gather(m):
        pltpu.make_async_copy(h2_hbm.at[c].at[idxc.at[m]], rows_v.at[m],
                              gsem[m]).wait()

    def wait_scatter(m):
        pltpu.make_async_copy(rows_v.at[m], acc.at[sdst.at[m]],
                              ssem[m]).wait()

    prefetch(0, 0)
    prefetch(1, 1)
    wait_prefetch(0, 0)
    issue_gather(0)

    def ring(i, carry):
        for b in range(3):
            j = 3 * i + b
            m = b
            m1 = (b + 1) % 3
            m2 = (b + 2) % 3
            j1 = j + 1
            j2 = j + 2

            @pl.when(j2 < N_CHUNK)
            def _():
                prefetch(j2, m2)

            @pl.when(j1 < N_CHUNK)
            def _():
                @pl.when(j1 >= 3)
                def _():
                    wait_scatter(m1)

                wait_prefetch(j1, m1)
                issue_gather(m1)

            wait_gather(m)
            for g in range(K // 16):
                sdst[m, pl.ds(g * 16, 16)] = dstc[m, pl.ds(g * 16, 16)]

            def sg(g, carry2):
                wg = ewc[m, pl.ds(g * 16, 16)]
                for l in range(16):
                    e = g * 16 + l
                    wv = _lane_splat(wg, l)
                    for gg in range(HALF // 16):
                        rows_v[m, e, pl.ds(gg * 16, 16)] = (
                            rows_v[m, e, pl.ds(gg * 16, 16)] * wv)
                return carry2

            lax.fori_loop(0, K // 16, sg, 0)
            pltpu.async_copy(rows_v.at[m], acc.at[sdst.at[m]], ssem[m],
                             add=True)
        return carry

    lax.fori_loop(0, N_CHUNK // 3, ring, 0)
    wait_scatter((N_CHUNK - 3) % 3)
    wait_scatter((N_CHUNK - 2) % 3)
    wait_scatter((N_CHUNK - 1) % 3)
    plsc.subcore_barrier()

    pltpu.sync_copy(b_hbm.at[pl.ds(c * HALF, HALF)], b_v)
    for kk in range(ROWS_TILE // FB):
        row0 = s * ROWS_TILE + kk * FB
        pltpu.sync_copy(acc.at[pl.ds(row0, FB)], rows_v.at[0, pl.ds(0, FB)])

        def fb(r, carry):
            for g in range(HALF // 16):
                v = (rows_v[0, r, pl.ds(g * 16, 16)]
                     + b_v[pl.ds(g * 16, 16)])
                rows_v[0, r, pl.ds(g * 16, 16)] = jnp.maximum(v, 0.0)
            return carry

        lax.fori_loop(0, FB, fb, 0)

        @pl.when(row0 < N_NODE)
        def _():
            pltpu.sync_copy(
                rows_v.at[0, pl.ds(0, FB)],
                out_hbm.at[pl.ds(row0, FB), pl.ds(c * HALF, HALF)])


def kernel(x, edge_index, edge_weight, W, b):
    src = edge_index[0]
    dst = edge_index[1]
    n_edges = src.shape[0]
    e_pad = N_TILE * E_TILE
    pad = e_pad - n_edges
    src_g = jnp.pad(src, (0, pad)).reshape(N_TILE, N_CHUNK, K)
    dst_g = jnp.pad(dst, (0, pad)).reshape(N_TILE, N_CHUNK, K)
    ew_g = jnp.pad(edge_weight, (0, pad)).reshape(N_TILE, N_CHUNK, K)
    x_pad = jnp.pad(x, ((0, N_PAD - x.shape[0]), (0, 0)))

    degp = _deg_kernel(src_g, ew_g)
    h2 = _matmul_scaled(x_pad, W, degp)
    return _agg_kernel(h2, src_g, ew_g, dst_g, b)
